# baseline (speedup 1.0000x reference)
import functools

import jax
import jax.numpy as jnp
from jax import lax
from jax.experimental import pallas as pl
from jax.experimental.pallas import tpu as pltpu
from jax.experimental.pallas import tpu_sc as plsc

_B, _T, _C, _H, _Wd = 2, 8, 96, 128, 128
_N = _H * _Wd
_K = _N // 2
_TCdim = _T * _C
_NWORK = 32
_SCAT_W = 128
_BLK = 512
_NB = _N // _BLK


def _leaky(y):
    return jnp.where(y >= 0, y, 0.01 * y)


def _router_scores(x, W1, b1, gamma, beta, W2, b2, W3, b3):
    B, T, C, H, W = x.shape
    N = H * W
    x_mean = x.mean(axis=1)
    x_max = x.max(axis=1)
    feat = jnp.concatenate([x_mean, x_max], axis=1)
    y = jnp.einsum('bchw,oc->bohw', feat, W1[:, :, 0, 0]) + b1[None, :, None, None]
    G = 4
    Cm = y.shape[1]
    yg = y.reshape(B, G, Cm // G, H, W)
    mu = yg.mean(axis=(2, 3, 4), keepdims=True)
    var = yg.var(axis=(2, 3, 4), keepdims=True)
    yg = (yg - mu) / jnp.sqrt(var + 1e-5)
    y = yg.reshape(B, Cm, H, W) * gamma[None, :, None, None] + beta[None, :, None, None]
    y = _leaky(y)
    y = jax.lax.conv_general_dilated(
        y, W2, window_strides=(1, 1), padding='SAME',
        dimension_numbers=('NCHW', 'OIHW', 'NCHW')) + b2[None, :, None, None]
    y = _leaky(y)
    y = jnp.einsum('bchw,oc->bohw', y, W3[:, :, 0, 0]) + b3[None, :, None, None]
    return jax.nn.sigmoid(y).reshape(B, N)


def _excl_prefix(a, U, Ls, ones):
    hi = lax.Precision.HIGHEST
    inc = lax.dot(a, U, precision=hi, preferred_element_type=jnp.float32)
    rowtot = lax.dot(a, ones, precision=hi, preferred_element_type=jnp.float32)
    rowoff = lax.dot(Ls, rowtot, precision=hi, preferred_element_type=jnp.float32)
    return inc + rowoff - a, rowoff


def _sel_body(s_ref, tgt_ref, mask_ref, cst_ref):
    b = pl.program_id(0)
    s = s_ref[0]
    bits = lax.bitcast_convert_type(s, jnp.int32)

    def bisect(i, carry):
        lo, hi = carry
        mid = lo + (hi - lo) // 2
        pred = jnp.sum((bits >= mid).astype(jnp.int32)) >= _K
        return (jnp.where(pred, mid, lo), jnp.where(pred, hi, mid))

    lo, hi = lax.fori_loop(0, 32, bisect, (jnp.int32(0), jnp.int32(0x7F000000)))

    gt = (bits > lo).astype(jnp.float32)
    eq = (bits == lo).astype(jnp.float32)
    m = jnp.sum(gt)
    r = jnp.float32(_K) - m

    r0 = lax.broadcasted_iota(jnp.int32, (128, 128), 0)
    c0 = lax.broadcasted_iota(jnp.int32, (128, 128), 1)
    U = (r0 <= c0).astype(jnp.float32)
    Ls = (c0 < r0).astype(jnp.float32)
    ones = jnp.ones((128, 1), jnp.float32)

    eqx, _ = _excl_prefix(eq, U, Ls, ones)
    keep = gt + eq * (eqx < r).astype(jnp.float32)
    kpx, rowoff = _excl_prefix(keep, U, Ls, ones)
    tgt = jnp.where(keep > 0, kpx + jnp.float32(_K) * b.astype(jnp.float32),
                    jnp.float32(_B * _K))
    tgt_ref[0] = tgt.astype(jnp.int32)

    rp = lax.broadcasted_iota(jnp.int32, (128, 8), 0)
    gp = lax.broadcasted_iota(jnp.int32, (128, 8), 1)
    P = (((rp // 16) == gp).astype(jnp.int32) << (rp % 16)).astype(jnp.float32)
    mask_ref[0] = lax.dot(keep, P, precision=lax.Precision.HIGHEST,
                          preferred_element_type=jnp.float32).astype(jnp.int32)
    cst_ref[0] = rowoff.astype(jnp.int32)


def _selection(scores_sq):
    return pl.pallas_call(
        _sel_body,
        grid=(_B,),
        in_specs=[pl.BlockSpec((1, 128, 128), lambda b: (b, 0, 0))],
        out_specs=[pl.BlockSpec((1, 128, 128), lambda b: (b, 0, 0)),
                   pl.BlockSpec((1, 128, 8), lambda b: (b, 0, 0)),
                   pl.BlockSpec((1, 128, 1), lambda b: (b, 0, 0))],
        out_shape=[jax.ShapeDtypeStruct((_B, 128, 128), jnp.int32),
                   jax.ShapeDtypeStruct((_B, 128, 8), jnp.int32),
                   jax.ShapeDtypeStruct((_B, 128, 1), jnp.int32)],
    )(scores_sq)


def _sc_scatter_body(tgt_hbm, outn_hbm, idx_v, val_v, sem):
    wid = lax.axis_index("s") * 2 + lax.axis_index("c")
    pltpu.sync_copy(tgt_hbm.at[wid], idx_v)
    base = wid * 1024
    nsub = jnp.where(wid >= _NWORK // _B, jnp.int32(_N), jnp.int32(0))
    start = base - nsub
    cps = []
    for j in range(8):
        if j >= 2:
            cps[j - 2].wait()
        buf = val_v.at[j % 2]

        def fill(r, _, _buf=buf, _off=start + j * 128):
            _buf[r, pl.ds(0, 16)] = jnp.full((16,), _off + r, jnp.int32)
            return 0

        lax.fori_loop(0, 128, fill, 0)
        cps.append(pltpu.async_copy(buf, outn_hbm.at[idx_v.at[j]], sem))
    cps[-2].wait()
    cps[-1].wait()


@functools.lru_cache(maxsize=None)
def _sc_scatter_kernel():
    mesh = plsc.VectorSubcoreMesh(core_axis_name="c", subcore_axis_name="s")
    return pl.kernel(
        _sc_scatter_body,
        out_type=jax.ShapeDtypeStruct((_B * _K + 1, _SCAT_W), jnp.int32),
        mesh=mesh,
        scratch_types=[pltpu.VMEM((8, 128), jnp.int32),
                       pltpu.VMEM((2, 128, _SCAT_W), jnp.int32),
                       pltpu.SemaphoreType.DMA],
    )


def _sc_scatter(tgt):
    return _sc_scatter_kernel()(tgt)


def _gt_body(tgt_s, x_ref, out_ref, stg, sem):
    nb = pl.program_id(0)
    stg[...] = x_ref[0].reshape(_TCdim, _BLK).T
    base = nb * _BLK

    def issue(r, _):
        t = tgt_s[base + r]
        pltpu.make_async_copy(stg.at[pl.ds(r, 1)],
                              out_ref.at[pl.ds(t, 1)], sem).start()
        return 0

    lax.fori_loop(0, _BLK, issue, 0)

    def drain(i, _):
        pltpu.make_async_copy(stg.at[pl.ds(0, 1)],
                              out_ref.at[pl.ds(0, 1)], sem).wait()
        return 0

    lax.fori_loop(0, _BLK, drain, 0)


def _gather_tc(tgt0, x4):
    grid_spec = pltpu.PrefetchScalarGridSpec(
        num_scalar_prefetch=1,
        grid=(_NB,),
        in_specs=[pl.BlockSpec((1, _T, _C, _BLK),
                               lambda n, t_ref: (0, 0, 0, n))],
        out_specs=pl.BlockSpec(memory_space=pl.ANY),
        scratch_shapes=[pltpu.VMEM((_BLK, _TCdim), jnp.float32),
                        pltpu.SemaphoreType.DMA],
    )
    return pl.pallas_call(
        _gt_body,
        grid_spec=grid_spec,
        out_shape=jax.ShapeDtypeStruct((_K + 1, _TCdim), jnp.float32),
    )(tgt0, x4)


def _tr_body(x_ref, out_ref):
    out_ref[:] = x_ref[0].reshape(_TCdim, _BLK).T


def _transpose1(x4):
    return pl.pallas_call(
        _tr_body,
        grid=(_NB,),
        in_specs=[pl.BlockSpec((1, _T, _C, _BLK), lambda n: (0, 0, 0, n))],
        out_specs=pl.BlockSpec((_BLK, _TCdim), lambda n: (n, 0)),
        out_shape=jax.ShapeDtypeStruct((_N, _TCdim), jnp.float32),
    )(x4)


_GC = 64
_NC1 = _K // (_NWORK * _GC)


def _sc_gather_body(idx_hbm, xflat_hbm, out_hbm, idx_v, rows_v, gsem, osem):
    wid = lax.axis_index("s") * 2 + lax.axis_index("c")
    pltpu.sync_copy(idx_hbm.at[wid], idx_v)
    base = wid * _NC1 * _GC
    g = [None] * _NC1
    o = [None] * _NC1
    g[0] = pltpu.async_copy(xflat_hbm.at[idx_v.at[0]], rows_v.at[0], gsem)
    for j in range(_NC1):
        cur = j % 2
        g[j].wait()
        if j >= 1:
            o[j - 1].wait()
        if j + 1 < _NC1:
            g[j + 1] = pltpu.async_copy(
                xflat_hbm.at[idx_v.at[j + 1]], rows_v.at[(j + 1) % 2], gsem)
        o[j] = pltpu.async_copy(
            rows_v.at[cur], out_hbm.at[pl.ds(base + j * _GC, _GC)], osem)
    o[_NC1 - 1].wait()


@functools.lru_cache(maxsize=None)
def _sc_gather_kernel():
    mesh = plsc.VectorSubcoreMesh(core_axis_name="c", subcore_axis_name="s")
    return pl.kernel(
        _sc_gather_body,
        out_type=jax.ShapeDtypeStruct((_K, _TCdim), jnp.float32),
        mesh=mesh,
        scratch_types=[pltpu.VMEM((_NC1, _GC), jnp.int32),
                       pltpu.VMEM((2, _GC, _TCdim), jnp.float32),
                       pltpu.SemaphoreType.DMA,
                       pltpu.SemaphoreType.DMA],
    )


def _sc_gather(idxg, x_flat):
    return _sc_gather_kernel()(idxg, x_flat)


def kernel(x, W1, b1, gamma, beta, W2, b2, W3, b3):
    B, T, C, H, W = x.shape
    scores = _router_scores(x, W1, b1, gamma, beta, W2, b2, W3, b3)

    tgt, mask, cst = _selection(scores.reshape(B, 128, 128))

    outn = _sc_scatter(tgt.reshape(_NWORK, 8, 128))
    top_idx = outn[:_B * _K, 0].reshape(B, _K)

    x4 = x.reshape(B, T, C, _N)
    tgt0 = jnp.minimum(tgt.reshape(_B * _N)[:_N], jnp.int32(_K))
    xs0 = _gather_tc(tgt0, x4[0:1])
    xflat1 = _transpose1(x4[1:2])
    idxg1 = outn[_K:_B * _K, 0].reshape(_NWORK, _NC1, _GC)
    xs1 = _sc_gather(idxg1, xflat1)
    x_sparse = jnp.concatenate([xs0[:_K], xs1], axis=0).reshape(B, _K, T, C)
    return (x_sparse, top_idx)


# speedup vs baseline: 1.4217x; 1.4217x over previous
import functools

import jax
import jax.numpy as jnp
from jax import lax
from jax.experimental import pallas as pl
from jax.experimental.pallas import tpu as pltpu
from jax.experimental.pallas import tpu_sc as plsc

_B, _T, _C, _H, _Wd = 2, 8, 96, 128, 128
_N = _H * _Wd
_K = _N // 2
_TCdim = _T * _C
_NWORK = 32
_SCAT_W = 128
_GCHUNK = 64
_NCHUNK = (_B * _K) // (_NWORK * _GCHUNK)


def _leaky(y):
    return jnp.where(y >= 0, y, 0.01 * y)


def _router_scores(x, W1, b1, gamma, beta, W2, b2, W3, b3):
    B, T, C, H, W = x.shape
    N = H * W
    x_mean = x.mean(axis=1)
    x_max = x.max(axis=1)
    feat = jnp.concatenate([x_mean, x_max], axis=1)
    y = jnp.einsum('bchw,oc->bohw', feat, W1[:, :, 0, 0]) + b1[None, :, None, None]
    G = 4
    Cm = y.shape[1]
    yg = y.reshape(B, G, Cm // G, H, W)
    mu = yg.mean(axis=(2, 3, 4), keepdims=True)
    var = yg.var(axis=(2, 3, 4), keepdims=True)
    yg = (yg - mu) / jnp.sqrt(var + 1e-5)
    y = yg.reshape(B, Cm, H, W) * gamma[None, :, None, None] + beta[None, :, None, None]
    y = _leaky(y)
    y = jax.lax.conv_general_dilated(
        y, W2, window_strides=(1, 1), padding='SAME',
        dimension_numbers=('NCHW', 'OIHW', 'NCHW')) + b2[None, :, None, None]
    y = _leaky(y)
    y = jnp.einsum('bchw,oc->bohw', y, W3[:, :, 0, 0]) + b3[None, :, None, None]
    return jax.nn.sigmoid(y).reshape(B, N)


def _tr_body(x_ref, out_ref):
    out_ref[0] = x_ref[0].reshape(_TCdim, 512).T


def _transpose(x4):
    nb = _N // 512
    return pl.pallas_call(
        _tr_body,
        grid=(_B, nb),
        in_specs=[pl.BlockSpec((1, _T, _C, 512), lambda b, n: (b, 0, 0, n))],
        out_specs=pl.BlockSpec((1, 512, _TCdim), lambda b, n: (b, n, 0)),
        out_shape=jax.ShapeDtypeStruct((_B, _N, _TCdim), jnp.float32),
    )(x4)


def _excl_prefix(a, U, Ls, ones):
    hi = lax.Precision.HIGHEST
    inc = lax.dot(a, U, precision=hi, preferred_element_type=jnp.float32)
    rowtot = lax.dot(a, ones, precision=hi, preferred_element_type=jnp.float32)
    rowoff = lax.dot(Ls, rowtot, precision=hi, preferred_element_type=jnp.float32)
    return inc + rowoff - a


def _sel_body(s_ref, tgt_ref):
    b = pl.program_id(0)
    s = s_ref[0]
    bits = lax.bitcast_convert_type(s, jnp.int32)

    def bisect(i, carry):
        lo, hi = carry
        mid = lo + (hi - lo) // 2
        pred = jnp.sum((bits >= mid).astype(jnp.int32)) >= _K
        return (jnp.where(pred, mid, lo), jnp.where(pred, hi, mid))

    lo, hi = lax.fori_loop(0, 32, bisect, (jnp.int32(0), jnp.int32(0x7F000000)))

    gt = (bits > lo).astype(jnp.float32)
    eq = (bits == lo).astype(jnp.float32)
    m = jnp.sum(gt)
    r = jnp.float32(_K) - m

    r0 = lax.broadcasted_iota(jnp.int32, (128, 128), 0)
    c0 = lax.broadcasted_iota(jnp.int32, (128, 128), 1)
    U = (r0 <= c0).astype(jnp.float32)
    Ls = (c0 < r0).astype(jnp.float32)
    ones = jnp.ones((128, 1), jnp.float32)

    eqx = _excl_prefix(eq, U, Ls, ones)
    keep = gt + eq * (eqx < r).astype(jnp.float32)
    kpx = _excl_prefix(keep, U, Ls, ones)
    tgt = jnp.where(keep > 0, kpx + jnp.float32(_K) * b.astype(jnp.float32),
                    jnp.float32(_B * _K))
    tgt_ref[0] = tgt.astype(jnp.int32)


def _selection(scores_sq):
    return pl.pallas_call(
        _sel_body,
        grid=(_B,),
        in_specs=[pl.BlockSpec((1, 128, 128), lambda b: (b, 0, 0))],
        out_specs=pl.BlockSpec((1, 128, 128), lambda b: (b, 0, 0)),
        out_shape=jax.ShapeDtypeStruct((_B, 128, 128), jnp.int32),
    )(scores_sq)


def _sc_scatter_body(tgt_hbm, outn_hbm, idx_v, val_v, sem):
    wid = lax.axis_index("s") * 2 + lax.axis_index("c")
    pltpu.sync_copy(tgt_hbm.at[wid], idx_v)
    base = wid * 1024
    nsub = jnp.where(wid >= _NWORK // _B, jnp.int32(_N), jnp.int32(0))
    start = base - nsub
    cps = []
    for j in range(8):
        if j >= 2:
            cps[j - 2].wait()
        buf = val_v.at[j % 2]

        def fill(r, _, _buf=buf, _off=start + j * 128):
            _buf[r, pl.ds(0, 16)] = jnp.full((16,), _off + r, jnp.int32)
            return 0

        lax.fori_loop(0, 128, fill, 0)
        cps.append(pltpu.async_copy(buf, outn_hbm.at[idx_v.at[j]], sem))
    cps[-2].wait()
    cps[-1].wait()


@functools.lru_cache(maxsize=None)
def _sc_scatter_kernel():
    mesh = plsc.VectorSubcoreMesh(core_axis_name="c", subcore_axis_name="s")
    return pl.kernel(
        _sc_scatter_body,
        out_type=jax.ShapeDtypeStruct((_B * _K + 1, _SCAT_W), jnp.int32),
        mesh=mesh,
        scratch_types=[pltpu.VMEM((8, 128), jnp.int32),
                       pltpu.VMEM((2, 128, _SCAT_W), jnp.int32),
                       pltpu.SemaphoreType.DMA],
    )


def _sc_scatter(tgt):
    return _sc_scatter_kernel()(tgt)


def _sc_gather_body(idx_hbm, xflat_hbm, out_hbm, idx_v, rows_v, gsem, osem):
    wid = lax.axis_index("s") * 2 + lax.axis_index("c")
    pltpu.sync_copy(idx_hbm.at[wid], idx_v)
    badd = jnp.where(wid >= _NWORK // _B, jnp.int32(_N), jnp.int32(0))
    for j in range(_NCHUNK):
        for s in range(_GCHUNK // 16):
            sl = (j, pl.ds(s * 16, 16))
            idx_v[sl] = idx_v[sl] + badd
    base = wid * (_NCHUNK * _GCHUNK)
    g = [None] * _NCHUNK
    o = [None] * _NCHUNK
    g[0] = pltpu.async_copy(xflat_hbm.at[idx_v.at[0]], rows_v.at[0], gsem)
    for j in range(_NCHUNK):
        cur = j % 2
        g[j].wait()
        if j >= 1:
            o[j - 1].wait()
        if j + 1 < _NCHUNK:
            g[j + 1] = pltpu.async_copy(
                xflat_hbm.at[idx_v.at[j + 1]], rows_v.at[(j + 1) % 2], gsem)
        o[j] = pltpu.async_copy(
            rows_v.at[cur], out_hbm.at[pl.ds(base + j * _GCHUNK, _GCHUNK)], osem)
    o[_NCHUNK - 1].wait()


@functools.lru_cache(maxsize=None)
def _sc_gather_kernel():
    mesh = plsc.VectorSubcoreMesh(core_axis_name="c", subcore_axis_name="s")
    return pl.kernel(
        _sc_gather_body,
        out_type=jax.ShapeDtypeStruct((_B * _K, _TCdim), jnp.float32),
        mesh=mesh,
        scratch_types=[pltpu.VMEM((_NCHUNK, _GCHUNK), jnp.int32),
                       pltpu.VMEM((2, _GCHUNK, _TCdim), jnp.float32),
                       pltpu.SemaphoreType.DMA,
                       pltpu.SemaphoreType.DMA],
    )


def _sc_gather(idxg, x_flat):
    return _sc_gather_kernel()(idxg, x_flat)


def kernel(x, W1, b1, gamma, beta, W2, b2, W3, b3):
    B, T, C, H, W = x.shape
    scores = _router_scores(x, W1, b1, gamma, beta, W2, b2, W3, b3)

    x_flat = _transpose(x.reshape(B, T, C, _N)).reshape(B * _N, _TCdim)
    tgt = _selection(scores.reshape(B, 128, 128)).reshape(_NWORK, 8, 128)

    outn = _sc_scatter(tgt)
    idx_flat = outn[:_B * _K, 0]
    xs = _sc_gather(idx_flat.reshape(_NWORK, _NCHUNK, _GCHUNK), x_flat)

    x_sparse = xs.reshape(B, _K, T, C)
    top_idx = idx_flat.reshape(B, _K)
    return (x_sparse, top_idx)
